# baseline (speedup 1.0000x reference)
import functools
import math

import jax
import jax.numpy as jnp
from jax import lax
from jax.experimental import pallas as pl
from jax.experimental.pallas import tpu as pltpu
from jax.experimental.pallas import tpu_sc as plsc

CLASS_DIM = 32
API_DIM = 64
CAT_DIM = CLASS_DIM + API_DIM
S32 = math.sqrt(float(CLASS_DIM))
S96 = math.sqrt(float(CAT_DIM))

NC = 2
NS = 16
NW = NC * NS
LANES = 16

B = 1024
L = 200
NGROUP = 8
NQ = 4
L_PER = L // NGROUP
B_CH = B // NQ
JT = B_CH // 128


V_API = 1000000
V_CH = 512


@functools.lru_cache(maxsize=None)
def _make_detile():
    def body(x_ref, y_ref, eye_ref):
        b = pl.program_id(0)

        @pl.when(b == 0)
        def _init():
            ii = lax.broadcasted_iota(jnp.int32, (V_CH, V_CH), 0)
            jj = lax.broadcasted_iota(jnp.int32, (V_CH, V_CH), 1)
            eye_ref[...] = (ii == jj).astype(jnp.float32)

        xt = lax.dot_general(
            eye_ref[...], x_ref[...],
            dimension_numbers=(((1,), (1,)), ((), ())),
            preferred_element_type=jnp.float32,
            precision=lax.Precision.HIGHEST)
        y_ref[...] = jnp.concatenate(
            [xt[0:V_CH // 2, :], xt[V_CH // 2:, :]], axis=1)

    n_blocks = (V_API + V_CH - 1) // V_CH
    return pl.pallas_call(
        body,
        grid=(n_blocks,),
        in_specs=[pl.BlockSpec((API_DIM, V_CH), lambda b: (0, b))],
        out_specs=pl.BlockSpec((V_CH // 2, 128), lambda b: (b, 0)),
        out_shape=jax.ShapeDtypeStruct((n_blocks * (V_CH // 2), 128),
                                       jnp.float32),
        scratch_shapes=[pltpu.VMEM((V_CH, V_CH), jnp.float32)],
    )


@functools.lru_cache(maxsize=None)
def _make_sc_kernel():
    mesh = plsc.VectorSubcoreMesh(core_axis_name="c", subcore_axis_name="s")

    @functools.partial(
        pl.kernel,
        mesh=mesh,
        compiler_params=pltpu.CompilerParams(
            use_tc_tiling_on_sc=False, needs_layout_passes=False),
        out_type=(
            jax.ShapeDtypeStruct((L, CLASS_DIM // 8, B // 128, 8, 128),
                                 jnp.float32),
            jax.ShapeDtypeStruct((L, CAT_DIM // 8, B // 128, 8, 128),
                                 jnp.float32),
        ),
        scratch_types=[
            pltpu.VMEM((B_CH,), jnp.int32),
            pltpu.VMEM((B_CH,), jnp.int32),
            pltpu.VMEM((B_CH,), jnp.int32),
            pltpu.VMEM((B_CH,), jnp.int32),
            pltpu.VMEM((B_CH,), jnp.int32),
            pltpu.VMEM((B_CH,), jnp.int32),
            pltpu.VMEM((B_CH, CLASS_DIM), jnp.float32),
            pltpu.VMEM((B_CH, CLASS_DIM), jnp.float32),
            pltpu.VMEM((B_CH, 128), jnp.float32),
            pltpu.VMEM((B_CH, 128), jnp.float32),
            pltpu.VMEM((CLASS_DIM // 8, JT, 8, 144), jnp.float32),
            pltpu.VMEM((CAT_DIM // 8, JT, 8, 144), jnp.float32),
            pltpu.SemaphoreType.DMA,
            pltpu.SemaphoreType.DMA,
        ],
    )
    def k(cls_idx, api_idx, cls_tab, api_tab2, ot1, ot2,
          cidx0, cidx1, gidx0, gidx1, offs0, offs1,
          clsv0, clsv1, apiv0, apiv1, cls_pl, cat_pl, sem1, sem2):
        cidx = (cidx0, cidx1)
        gidx = (gidx0, gidx1)
        offs = (offs0, offs1)
        clsv = (clsv0, clsv1)
        apiv = (apiv0, apiv1)

        wid = lax.axis_index("s") * NC + lax.axis_index("c")
        g = wid % NGROUP
        q = wid // NGROUP
        l0 = g * L_PER
        b0 = q * B_CH

        iota = lax.iota(jnp.int32, LANES)
        cls_i, cls_s, api_i, api_s = [], [], [], []
        for h in range(CLASS_DIM // LANES):
            d = iota + h * LANES
            cls_i.append(lax.shift_right_logical(d, 3))
            cls_s.append(lax.bitwise_and(d, 7))
        for h in range(API_DIM // LANES):
            d = iota + CLASS_DIM + h * LANES
            api_i.append(lax.shift_right_logical(d, 3))
            api_s.append(lax.bitwise_and(d, 7))

        def stage(j, s):
            pltpu.sync_copy(cls_idx.at[l0 + j, pl.ds(b0, B_CH)], cidx[s])
            pltpu.sync_copy(api_idx.at[l0 + j, pl.ds(b0, B_CH)], gidx[s])

            def prep(t, carry):
                a = gidx[s][pl.ds(t * LANES, LANES)]
                offs[s][pl.ds(t * LANES, LANES)] = (
                    lax.bitwise_and(lax.shift_right_logical(a, 8), 1)
                    * API_DIM)
                gidx[s][pl.ds(t * LANES, LANES)] = (
                    lax.bitwise_or(
                        lax.shift_left(lax.shift_right_logical(a, 9), 8),
                        lax.bitwise_and(a, 255)))
                return carry
            lax.fori_loop(0, B_CH // LANES, prep, 0)

            pltpu.async_copy(cls_tab.at[cidx[s]], clsv[s], sem1)
            pltpu.async_copy(api_tab2.at[gidx[s]], apiv[s], sem2)

        def wait_gathers(s):
            pltpu.make_async_copy(cls_tab.at[cidx[s]], clsv[s], sem1).wait()
            pltpu.make_async_copy(api_tab2.at[gidx[s]], apiv[s], sem2).wait()

        def compute_and_emit(j, s):
            def row_body(r, carry):
                jj = lax.shift_right_logical(r, 7)
                c = lax.bitwise_and(r, 127)
                r_v = jnp.broadcast_to(r, (LANES,))
                jj_v = jnp.broadcast_to(jj, (LANES,))
                c_v = jnp.broadcast_to(c, (LANES,))
                off16 = plsc.load_gather(offs[s], [r_v])
                for h in range(CLASS_DIM // LANES):
                    v = clsv[s][r, pl.ds(h * LANES, LANES)]
                    plsc.store_scatter(cat_pl, [cls_i[h], jj_v, cls_s[h], c_v],
                                       v * S96)
                    plsc.store_scatter(cls_pl, [cls_i[h], jj_v, cls_s[h], c_v],
                                       v * S32)
                for h in range(API_DIM // LANES):
                    v = plsc.load_gather(apiv[s],
                                         [r_v, off16 + (h * LANES) + iota])
                    plsc.store_scatter(cat_pl, [api_i[h], jj_v, api_s[h], c_v],
                                       v * S96)
                return carry

            lax.fori_loop(0, B_CH, row_body, 0)

            pltpu.sync_copy(cls_pl.at[:, :, :, pl.ds(0, 128)],
                            ot1.at[l0 + j, :, pl.ds(q * JT, JT), :, :])
            pltpu.sync_copy(cat_pl.at[:, :, :, pl.ds(0, 128)],
                            ot2.at[l0 + j, :, pl.ds(q * JT, JT), :, :])

        stage(0, 0)

        def two_chunks(t, carry):
            jb = 2 * t
            wait_gathers(0)
            stage(jb + 1, 1)
            compute_and_emit(jb, 0)
            wait_gathers(1)
            stage(jb + 2, 0)
            compute_and_emit(jb + 1, 1)
            return carry

        lax.fori_loop(0, (L_PER - 1) // 2, two_chunks, 0)
        wait_gathers(0)
        compute_and_emit(L_PER - 1, 0)

    return k


def kernel(class_seq, api_seq, class_table, api_table):
    cls_idx = class_seq.T.astype(jnp.int32)
    api_idx = api_seq.T.astype(jnp.int32)
    api_pairs = _make_detile()(api_table.T)
    ot1, ot2 = _make_sc_kernel()(cls_idx, api_idx, class_table, api_pairs)
    out1 = ot1.transpose(2, 4, 0, 1, 3).reshape(B, L, CLASS_DIM)
    out2 = ot2.transpose(2, 4, 0, 1, 3).reshape(B, L, CAT_DIM)
    return (out1, out2)


# speedup vs baseline: 2.0301x; 2.0301x over previous
import functools
import math

import jax
import jax.numpy as jnp
from jax import lax
from jax.experimental import pallas as pl
from jax.experimental.pallas import tpu as pltpu
from jax.experimental.pallas import tpu_sc as plsc

CLASS_DIM = 32
API_DIM = 64
CAT_DIM = CLASS_DIM + API_DIM
S32 = math.sqrt(float(CLASS_DIM))
S96 = math.sqrt(float(CAT_DIM))

NC = 2
NS = 16
NW = NC * NS
LANES = 16

B = 1024
L = 200
NGROUP = 4
NQ = 8
L_PER = L // NGROUP
B_CH = B // NQ
JT = B_CH // 128


V_API = 1000000
V_CH = 512


@functools.lru_cache(maxsize=None)
def _make_detile():
    def body(x_ref, y_ref, eye_ref):
        b = pl.program_id(0)

        @pl.when(b == 0)
        def _init():
            ii = lax.broadcasted_iota(jnp.int32, (V_CH, V_CH), 0)
            jj = lax.broadcasted_iota(jnp.int32, (V_CH, V_CH), 1)
            eye_ref[...] = (ii == jj).astype(jnp.float32)

        xt = lax.dot_general(
            eye_ref[...], x_ref[...],
            dimension_numbers=(((1,), (1,)), ((), ())),
            preferred_element_type=jnp.float32,
            precision=lax.Precision.HIGHEST)
        y_ref[...] = jnp.concatenate(
            [xt[0:V_CH // 2, :], xt[V_CH // 2:, :]], axis=1)

    n_blocks = (V_API + V_CH - 1) // V_CH
    return pl.pallas_call(
        body,
        grid=(n_blocks,),
        in_specs=[pl.BlockSpec((API_DIM, V_CH), lambda b: (0, b))],
        out_specs=pl.BlockSpec((V_CH // 2, 128), lambda b: (b, 0)),
        out_shape=jax.ShapeDtypeStruct((n_blocks * (V_CH // 2), 128),
                                       jnp.float32),
        scratch_shapes=[pltpu.VMEM((V_CH, V_CH), jnp.float32)],
    )


@functools.lru_cache(maxsize=None)
def _make_sc_kernel():
    mesh = plsc.VectorSubcoreMesh(core_axis_name="c", subcore_axis_name="s")

    @functools.partial(
        pl.kernel,
        mesh=mesh,
        compiler_params=pltpu.CompilerParams(
            use_tc_tiling_on_sc=False, needs_layout_passes=False),
        out_type=(
            jax.ShapeDtypeStruct((L, CLASS_DIM // 8, B // 128, 8, 128),
                                 jnp.float32),
            jax.ShapeDtypeStruct((L, CAT_DIM // 8, B // 128, 8, 128),
                                 jnp.float32),
        ),
        scratch_types=[
            pltpu.VMEM((B_CH,), jnp.int32),
            pltpu.VMEM((B_CH,), jnp.int32),
            pltpu.VMEM((B_CH,), jnp.int32),
            pltpu.VMEM((B_CH,), jnp.int32),
            pltpu.VMEM((B_CH,), jnp.int32),
            pltpu.VMEM((B_CH,), jnp.int32),
            pltpu.VMEM((B_CH, CLASS_DIM), jnp.float32),
            pltpu.VMEM((B_CH, CLASS_DIM), jnp.float32),
            pltpu.VMEM((B_CH, 32), jnp.float32),
            pltpu.VMEM((B_CH, 32), jnp.float32),
            pltpu.VMEM((B_CH, 32), jnp.float32),
            pltpu.VMEM((B_CH, 32), jnp.float32),
            pltpu.VMEM((CLASS_DIM // 8, JT, 8, 128), jnp.float32),
            pltpu.VMEM((CAT_DIM // 8, JT, 8, 128), jnp.float32),
            pltpu.SemaphoreType.DMA,
            pltpu.SemaphoreType.DMA,
        ],
    )
    def k(cls_idx, api_idx, cls_tab, api_tab4, ot1, ot2,
          cidx0, cidx1, glo0, glo1, ghi0, ghi1,
          clsv0, clsv1, alov0, alov1, ahiv0, ahiv1, cls_pl, cat_pl,
          sem1, sem2):
        cidx = (cidx0, cidx1)
        glo = (glo0, glo1)
        ghi = (ghi0, ghi1)
        clsv = (clsv0, clsv1)
        alov = (alov0, alov1)
        ahiv = (ahiv0, ahiv1)

        wid = lax.axis_index("s") * NC + lax.axis_index("c")
        g = wid % NGROUP
        q = wid // NGROUP
        l0 = g * L_PER
        b0 = q * B_CH

        iota = lax.iota(jnp.int32, LANES)

        def stage(j, s):
            pltpu.sync_copy(cls_idx.at[l0 + j, pl.ds(b0, B_CH)], cidx[s])
            pltpu.sync_copy(api_idx.at[l0 + j, pl.ds(b0, B_CH)], glo[s])

            def prep(t, carry):
                a = glo[s][pl.ds(t * LANES, LANES)]
                glo[s][pl.ds(t * LANES, LANES)] = a + a
                ghi[s][pl.ds(t * LANES, LANES)] = a + a + 1
                return carry
            lax.fori_loop(0, B_CH // LANES, prep, 0)

            pltpu.async_copy(cls_tab.at[cidx[s]], clsv[s], sem1)
            pltpu.async_copy(api_tab4.at[glo[s]], alov[s], sem2)
            pltpu.async_copy(api_tab4.at[ghi[s]], ahiv[s], sem2)

        def wait_gathers(s):
            pltpu.make_async_copy(cls_tab.at[cidx[s]], clsv[s], sem1).wait()
            pltpu.make_async_copy(api_tab4.at[glo[s]], alov[s], sem2).wait()
            pltpu.make_async_copy(api_tab4.at[ghi[s]], ahiv[s], sem2).wait()

        def compute_and_emit(j, s):
            def rblk_body(t, carry):
                r0 = t * LANES
                jj = lax.shift_right_logical(r0, 7)
                c0 = lax.bitwise_and(r0, 127)
                rows = iota + r0
                for d in range(CLASS_DIM):
                    dv = jnp.broadcast_to(d, (LANES,))
                    v = plsc.load_gather(clsv[s], [rows, dv])
                    cat_pl[d // 8, jj, d % 8, pl.ds(c0, LANES)] = v * S96
                    cls_pl[d // 8, jj, d % 8, pl.ds(c0, LANES)] = v * S32
                for d in range(32):
                    dv = jnp.broadcast_to(d, (LANES,))
                    v = plsc.load_gather(alov[s], [rows, dv])
                    cat_pl[(CLASS_DIM + d) // 8, jj, d % 8,
                           pl.ds(c0, LANES)] = v * S96
                for d in range(32):
                    dv = jnp.broadcast_to(d, (LANES,))
                    v = plsc.load_gather(ahiv[s], [rows, dv])
                    cat_pl[(CLASS_DIM + 32 + d) // 8, jj, d % 8,
                           pl.ds(c0, LANES)] = v * S96
                return carry

            lax.fori_loop(0, B_CH // LANES, rblk_body, 0)

            pltpu.sync_copy(cls_pl,
                            ot1.at[l0 + j, :, pl.ds(q * JT, JT), :, :])
            pltpu.sync_copy(cat_pl,
                            ot2.at[l0 + j, :, pl.ds(q * JT, JT), :, :])

        stage(0, 0)

        def two_chunks(t, carry):
            jb = 2 * t
            wait_gathers(0)
            stage(jb + 1, 1)
            compute_and_emit(jb, 0)
            wait_gathers(1)
            stage(jb + 2, 0)
            compute_and_emit(jb + 1, 1)
            return carry

        lax.fori_loop(0, (L_PER - 1) // 2, two_chunks, 0)
        if L_PER % 2:
            wait_gathers(0)
            compute_and_emit(L_PER - 1, 0)
        else:
            wait_gathers(0)
            stage(L_PER - 1, 1)
            compute_and_emit(L_PER - 2, 0)
            wait_gathers(1)
            compute_and_emit(L_PER - 1, 1)

    return k


def kernel(class_seq, api_seq, class_table, api_table):
    cls_idx = class_seq.T.astype(jnp.int32)
    api_idx = api_seq.T.astype(jnp.int32)
    api_pairs = api_table.reshape(-1, 32)
    ot1, ot2 = _make_sc_kernel()(cls_idx, api_idx, class_table, api_pairs)
    out1 = ot1.transpose(2, 4, 0, 1, 3).reshape(B, L, CLASS_DIM)
    out2 = ot2.transpose(2, 4, 0, 1, 3).reshape(B, L, CAT_DIM)
    return (out1, out2)


# speedup vs baseline: 2.5649x; 1.2634x over previous
import functools
import math

import jax
import jax.numpy as jnp
from jax import lax
from jax.experimental import pallas as pl
from jax.experimental.pallas import tpu as pltpu
from jax.experimental.pallas import tpu_sc as plsc

CLASS_DIM = 32
API_DIM = 64
CAT_DIM = CLASS_DIM + API_DIM
S32 = math.sqrt(float(CLASS_DIM))
S96 = math.sqrt(float(CAT_DIM))

NC = 2
NS = 16
NW = NC * NS
LANES = 16

B = 1024
L = 200
NGROUP = 8
NQ = 4
L_PER = L // NGROUP
B_CH = B // NQ
JT = B_CH // 128


V_API = 1000000
V_CH = 512


@functools.lru_cache(maxsize=None)
def _make_detile():
    def body(x_ref, y_ref, eye_ref):
        b = pl.program_id(0)

        @pl.when(b == 0)
        def _init():
            ii = lax.broadcasted_iota(jnp.int32, (V_CH, V_CH), 0)
            jj = lax.broadcasted_iota(jnp.int32, (V_CH, V_CH), 1)
            eye_ref[...] = (ii == jj).astype(jnp.float32)

        xt = lax.dot_general(
            eye_ref[...], x_ref[...],
            dimension_numbers=(((1,), (1,)), ((), ())),
            preferred_element_type=jnp.float32,
            precision=lax.Precision.HIGHEST)
        y_ref[...] = jnp.concatenate(
            [xt[0:V_CH // 2, :], xt[V_CH // 2:, :]], axis=1)

    n_blocks = (V_API + V_CH - 1) // V_CH
    return pl.pallas_call(
        body,
        grid=(n_blocks,),
        in_specs=[pl.BlockSpec((API_DIM, V_CH), lambda b: (0, b))],
        out_specs=pl.BlockSpec((V_CH // 2, 128), lambda b: (b, 0)),
        out_shape=jax.ShapeDtypeStruct((n_blocks * (V_CH // 2), 128),
                                       jnp.float32),
        scratch_shapes=[pltpu.VMEM((V_CH, V_CH), jnp.float32)],
    )


@functools.lru_cache(maxsize=None)
def _make_sc_kernel():
    mesh = plsc.VectorSubcoreMesh(core_axis_name="c", subcore_axis_name="s")

    @functools.partial(
        pl.kernel,
        mesh=mesh,
        compiler_params=pltpu.CompilerParams(
            use_tc_tiling_on_sc=False, needs_layout_passes=False),
        out_type=(
            jax.ShapeDtypeStruct((L, CLASS_DIM // 8, B // 128, 8, 128),
                                 jnp.float32),
            jax.ShapeDtypeStruct((L, CAT_DIM // 8, B // 128, 8, 128),
                                 jnp.float32),
        ),
        scratch_types=[
            pltpu.VMEM((B_CH,), jnp.int32),
            pltpu.VMEM((B_CH,), jnp.int32),
            pltpu.VMEM((B_CH,), jnp.int32),
            pltpu.VMEM((B_CH,), jnp.int32),
            pltpu.VMEM((B_CH,), jnp.int32),
            pltpu.VMEM((B_CH,), jnp.int32),
            pltpu.VMEM((B_CH, CLASS_DIM), jnp.float32),
            pltpu.VMEM((B_CH, CLASS_DIM), jnp.float32),
            pltpu.VMEM((B_CH, 32), jnp.float32),
            pltpu.VMEM((B_CH, 32), jnp.float32),
            pltpu.VMEM((B_CH, 32), jnp.float32),
            pltpu.VMEM((B_CH, 32), jnp.float32),
            pltpu.VMEM((CLASS_DIM // 8, JT, 8, 144), jnp.float32),
            pltpu.VMEM((CAT_DIM // 8, JT, 8, 144), jnp.float32),
            pltpu.SemaphoreType.DMA,
            pltpu.SemaphoreType.DMA,
        ],
    )
    def k(cls_idx, api_idx, cls_tab, api_tab4, ot1, ot2,
          cidx0, cidx1, glo0, glo1, ghi0, ghi1,
          clsv0, clsv1, alov0, alov1, ahiv0, ahiv1, cls_pl, cat_pl,
          sem1, sem2):
        cidx = (cidx0, cidx1)
        glo = (glo0, glo1)
        ghi = (ghi0, ghi1)
        clsv = (clsv0, clsv1)
        alov = (alov0, alov1)
        ahiv = (ahiv0, ahiv1)

        wid = lax.axis_index("s") * NC + lax.axis_index("c")
        g = wid % NGROUP
        q = wid // NGROUP
        l0 = g * L_PER
        b0 = q * B_CH

        iota = lax.iota(jnp.int32, LANES)
        pl_i, pl_s = [], []
        for h in range(CAT_DIM // LANES):
            d = iota + h * LANES
            pl_i.append(lax.shift_right_logical(d, 3))
            pl_s.append(lax.bitwise_and(d, 7))

        def stage(j, s):
            pltpu.sync_copy(cls_idx.at[l0 + j, pl.ds(b0, B_CH)], cidx[s])
            pltpu.sync_copy(api_idx.at[l0 + j, pl.ds(b0, B_CH)], glo[s])

            def prep(t, carry):
                a = glo[s][pl.ds(t * LANES, LANES)]
                glo[s][pl.ds(t * LANES, LANES)] = a + a
                ghi[s][pl.ds(t * LANES, LANES)] = a + a + 1
                return carry
            lax.fori_loop(0, B_CH // LANES, prep, 0)

            pltpu.async_copy(cls_tab.at[cidx[s]], clsv[s], sem1)
            pltpu.async_copy(api_tab4.at[glo[s]], alov[s], sem2)
            pltpu.async_copy(api_tab4.at[ghi[s]], ahiv[s], sem2)

        def wait_gathers(s):
            pltpu.make_async_copy(cls_tab.at[cidx[s]], clsv[s], sem1).wait()
            pltpu.make_async_copy(api_tab4.at[glo[s]], alov[s], sem2).wait()
            pltpu.make_async_copy(api_tab4.at[ghi[s]], ahiv[s], sem2).wait()

        def compute_and_emit(j, s):
            def row_body(r, carry):
                jj = lax.shift_right_logical(r, 7)
                c = lax.bitwise_and(r, 127)
                jj_v = jnp.broadcast_to(jj, (LANES,))
                c_v = jnp.broadcast_to(c, (LANES,))
                for h in range(CLASS_DIM // LANES):
                    v = clsv[s][r, pl.ds(h * LANES, LANES)]
                    plsc.store_scatter(cat_pl, [pl_i[h], jj_v, pl_s[h], c_v],
                                       v * S96)
                    plsc.store_scatter(cls_pl, [pl_i[h], jj_v, pl_s[h], c_v],
                                       v * S32)
                for h in range(2):
                    v = alov[s][r, pl.ds(h * LANES, LANES)]
                    plsc.store_scatter(cat_pl,
                                       [pl_i[2 + h], jj_v, pl_s[2 + h], c_v],
                                       v * S96)
                for h in range(2):
                    v = ahiv[s][r, pl.ds(h * LANES, LANES)]
                    plsc.store_scatter(cat_pl,
                                       [pl_i[4 + h], jj_v, pl_s[4 + h], c_v],
                                       v * S96)
                return carry

            lax.fori_loop(0, B_CH, row_body, 0)

            pltpu.sync_copy(cls_pl.at[:, :, :, pl.ds(0, 128)],
                            ot1.at[l0 + j, :, pl.ds(q * JT, JT), :, :])
            pltpu.sync_copy(cat_pl.at[:, :, :, pl.ds(0, 128)],
                            ot2.at[l0 + j, :, pl.ds(q * JT, JT), :, :])

        stage(0, 0)

        def two_chunks(t, carry):
            jb = 2 * t
            wait_gathers(0)
            stage(jb + 1, 1)
            compute_and_emit(jb, 0)
            wait_gathers(1)
            stage(jb + 2, 0)
            compute_and_emit(jb + 1, 1)
            return carry

        lax.fori_loop(0, (L_PER - 1) // 2, two_chunks, 0)
        if L_PER % 2:
            wait_gathers(0)
            compute_and_emit(L_PER - 1, 0)
        else:
            wait_gathers(0)
            stage(L_PER - 1, 1)
            compute_and_emit(L_PER - 2, 0)
            wait_gathers(1)
            compute_and_emit(L_PER - 1, 1)

    return k


def kernel(class_seq, api_seq, class_table, api_table):
    cls_idx = class_seq.T.astype(jnp.int32)
    api_idx = api_seq.T.astype(jnp.int32)
    api_pairs = api_table.reshape(-1, 32)
    ot1, ot2 = _make_sc_kernel()(cls_idx, api_idx, class_table, api_pairs)
    out1 = ot1.transpose(2, 4, 0, 1, 3).reshape(B, L, CLASS_DIM)
    out2 = ot2.transpose(2, 4, 0, 1, 3).reshape(B, L, CAT_DIM)
    return (out1, out2)


# speedup vs baseline: 2.5686x; 1.0014x over previous
import functools
import math

import jax
import jax.numpy as jnp
from jax import lax
from jax.experimental import pallas as pl
from jax.experimental.pallas import tpu as pltpu
from jax.experimental.pallas import tpu_sc as plsc

CLASS_DIM = 32
API_DIM = 64
CAT_DIM = CLASS_DIM + API_DIM
S32 = math.sqrt(float(CLASS_DIM))
S96 = math.sqrt(float(CAT_DIM))

NC = 2
NS = 16
NW = NC * NS
LANES = 16

B = 1024
L = 200
NGROUP = 8
NQ = 4
L_PER = L // NGROUP
B_CH = B // NQ
JT = B_CH // 128


@functools.lru_cache(maxsize=None)
def _make_sc_kernel():
    mesh = plsc.VectorSubcoreMesh(core_axis_name="c", subcore_axis_name="s")

    @functools.partial(
        pl.kernel,
        mesh=mesh,
        compiler_params=pltpu.CompilerParams(
            use_tc_tiling_on_sc=False, needs_layout_passes=False),
        out_type=(
            jax.ShapeDtypeStruct((L, CLASS_DIM // 8, B // 128, 8, 128),
                                 jnp.float32),
            jax.ShapeDtypeStruct((L, CAT_DIM // 8, B // 128, 8, 128),
                                 jnp.float32),
        ),
        scratch_types=[
            pltpu.VMEM((B_CH,), jnp.int32),
            pltpu.VMEM((B_CH,), jnp.int32),
            pltpu.VMEM((B_CH,), jnp.int32),
            pltpu.VMEM((B_CH,), jnp.int32),
            pltpu.VMEM((B_CH,), jnp.int32),
            pltpu.VMEM((B_CH,), jnp.int32),
            pltpu.VMEM((B_CH, CLASS_DIM), jnp.float32),
            pltpu.VMEM((B_CH, CLASS_DIM), jnp.float32),
            pltpu.VMEM((B_CH, 32), jnp.float32),
            pltpu.VMEM((B_CH, 32), jnp.float32),
            pltpu.VMEM((B_CH, 32), jnp.float32),
            pltpu.VMEM((B_CH, 32), jnp.float32),
            pltpu.VMEM((CLASS_DIM // 8, JT, 8, 144), jnp.float32),
            pltpu.VMEM((CAT_DIM // 8, JT, 8, 144), jnp.float32),
            pltpu.SemaphoreType.DMA,
            pltpu.SemaphoreType.DMA,
        ],
    )
    def k(cls_idx, api_idx, cls_tab, api_tab4, ot1, ot2,
          cidx0, cidx1, glo0, glo1, ghi0, ghi1,
          clsv0, clsv1, alov0, alov1, ahiv0, ahiv1, cls_pl, cat_pl,
          sem1, sem2):
        cidx = (cidx0, cidx1)
        glo = (glo0, glo1)
        ghi = (ghi0, ghi1)
        clsv = (clsv0, clsv1)
        alov = (alov0, alov1)
        ahiv = (ahiv0, ahiv1)

        wid = lax.axis_index("s") * NC + lax.axis_index("c")
        g = wid % NGROUP
        q = wid // NGROUP
        l0 = g * L_PER
        b0 = q * B_CH

        iota = lax.iota(jnp.int32, LANES)
        pl_i, pl_s = [], []
        for h in range(CAT_DIM // LANES):
            d = iota + h * LANES
            pl_i.append(lax.shift_right_logical(d, 3))
            pl_s.append(lax.bitwise_and(d, 7))

        def stage(j, s):
            pltpu.sync_copy(cls_idx.at[l0 + j, pl.ds(b0, B_CH)], cidx[s])
            pltpu.sync_copy(api_idx.at[l0 + j, pl.ds(b0, B_CH)], glo[s])

            def prep(t, carry):
                a = glo[s][pl.ds(t * LANES, LANES)]
                glo[s][pl.ds(t * LANES, LANES)] = a + a
                ghi[s][pl.ds(t * LANES, LANES)] = a + a + 1
                return carry
            lax.fori_loop(0, B_CH // LANES, prep, 0)

            pltpu.async_copy(cls_tab.at[cidx[s]], clsv[s], sem1)
            pltpu.async_copy(api_tab4.at[glo[s]], alov[s], sem2)
            pltpu.async_copy(api_tab4.at[ghi[s]], ahiv[s], sem2)

        def wait_gathers(s):
            pltpu.make_async_copy(cls_tab.at[cidx[s]], clsv[s], sem1).wait()
            pltpu.make_async_copy(api_tab4.at[glo[s]], alov[s], sem2).wait()
            pltpu.make_async_copy(api_tab4.at[ghi[s]], ahiv[s], sem2).wait()

        def compute_and_emit(j, s):
            def row_body(r, carry):
                jj = lax.shift_right_logical(r, 7)
                c = lax.bitwise_and(r, 127)
                jj_v = jnp.broadcast_to(jj, (LANES,))
                c_v = jnp.broadcast_to(c, (LANES,))
                for h in range(CLASS_DIM // LANES):
                    v = clsv[s][r, pl.ds(h * LANES, LANES)]
                    plsc.store_scatter(cat_pl, [pl_i[h], jj_v, pl_s[h], c_v],
                                       v * S96)
                    plsc.store_scatter(cls_pl, [pl_i[h], jj_v, pl_s[h], c_v],
                                       v * S32)
                for h in range(2):
                    v = alov[s][r, pl.ds(h * LANES, LANES)]
                    plsc.store_scatter(cat_pl,
                                       [pl_i[2 + h], jj_v, pl_s[2 + h], c_v],
                                       v * S96)
                for h in range(2):
                    v = ahiv[s][r, pl.ds(h * LANES, LANES)]
                    plsc.store_scatter(cat_pl,
                                       [pl_i[4 + h], jj_v, pl_s[4 + h], c_v],
                                       v * S96)
                return carry

            lax.fori_loop(0, B_CH, row_body, 0)

            pltpu.sync_copy(cls_pl.at[:, :, :, pl.ds(0, 128)],
                            ot1.at[l0 + j, :, pl.ds(q * JT, JT), :, :])
            pltpu.sync_copy(cat_pl.at[:, :, :, pl.ds(0, 128)],
                            ot2.at[l0 + j, :, pl.ds(q * JT, JT), :, :])

        stage(0, 0)

        def two_chunks(t, carry):
            jb = 2 * t
            wait_gathers(0)
            stage(jb + 1, 1)
            compute_and_emit(jb, 0)
            wait_gathers(1)
            stage(jb + 2, 0)
            compute_and_emit(jb + 1, 1)
            return carry

        lax.fori_loop(0, (L_PER - 1) // 2, two_chunks, 0)
        if L_PER % 2:
            wait_gathers(0)
            compute_and_emit(L_PER - 1, 0)
        else:
            wait_gathers(0)
            stage(L_PER - 1, 1)
            compute_and_emit(L_PER - 2, 0)
            wait_gathers(1)
            compute_and_emit(L_PER - 1, 1)

    return k


def kernel(class_seq, api_seq, class_table, api_table):
    cls_idx = class_seq.T.astype(jnp.int32)
    api_idx = api_seq.T.astype(jnp.int32)
    api_pairs = api_table.reshape(-1, 32)
    ot1, ot2 = _make_sc_kernel()(cls_idx, api_idx, class_table, api_pairs)
    out1 = ot1.transpose(2, 4, 0, 1, 3).reshape(B, L, CLASS_DIM)
    out2 = ot2.transpose(2, 4, 0, 1, 3).reshape(B, L, CAT_DIM)
    return (out1, out2)
